# baseline (speedup 1.0000x reference)
import jax
import jax.numpy as jnp
from jax import lax
from jax.experimental import pallas as pl
from jax.experimental.pallas import tpu as pltpu
from jax.experimental.pallas import tpu_sc as plsc

N = 10000
E = 320000
IN_FEAT = 128
OUT_FEAT = 128
NUM_RELS = 32
NUM_BASES = 8

NC = 2
NS = 16
NW = NC * NS
CHUNK = 128
NBUF = 2
IBUF = 2 * NBUF
CPW = -(-E // (CHUNK * NW * IBUF)) * IBUF
NCHUNKS = CPW * NW
E_PAD = NCHUNKS * CHUNK
NPAD = 10240
ROWS_PER_TILE = NPAD // NS
BN = 1000


def _xw_body(w_comp_ref, weight_ref, x_ref, out_ref):
    r = pl.program_id(1)
    w = w_comp_ref[r, 0] * weight_ref[0]
    for b in range(1, NUM_BASES):
        w = w + w_comp_ref[r, b] * weight_ref[b]
    out_ref[0] = jnp.dot(x_ref[...], w, preferred_element_type=jnp.float32)


def _sc_body(xw_hbm, idd_hbm, zeros_hbm, out_hbm,
             idd0, idd1, idd2, idd3, rows0, rows1, h_shared,
             isem0, isem1, isem2, isem3, gsem0, gsem1):
    idd = (idd0, idd1, idd2, idd3)
    isem = (isem0, isem1, isem2, isem3)
    rows = (rows0, rows1)
    gsem = (gsem0, gsem1)
    c = lax.axis_index("c")
    s = lax.axis_index("s")
    wid = s * NC + c
    tile_base = s * ROWS_PER_TILE
    pltpu.sync_copy(zeros_hbm, h_shared.at[pl.ds(tile_base, ROWS_PER_TILE)])
    plsc.subcore_barrier()

    for j in range(IBUF):
        pltpu.async_copy(idd_hbm.at[wid, j], idd[j], isem[j])
    for j in range(NBUF):
        pltpu.make_async_copy(idd_hbm.at[wid, j], idd[j], isem[j]).wait()
        pltpu.async_copy(xw_hbm.at[idd[j].at[0]], rows[j], gsem[j])

    def step(j0, carry):
        for u in range(IBUF):
            j = j0 * IBUF + u
            ib, rb = u, u % NBUF
            pltpu.make_async_copy(xw_hbm.at[idd[ib].at[0]], rows[rb],
                                  gsem[rb]).wait()
            pltpu.sync_copy(rows[rb], h_shared.at[idd[ib].at[1]], add=True)

            @pl.when(j + IBUF < CPW)
            def _():
                pltpu.async_copy(idd_hbm.at[wid, j + IBUF], idd[ib], isem[ib])

            @pl.when(j + NBUF < CPW)
            def _():
                ib2 = (u + NBUF) % IBUF
                pltpu.make_async_copy(idd_hbm.at[wid, j + NBUF], idd[ib2],
                                      isem[ib2]).wait()
                pltpu.async_copy(xw_hbm.at[idd[ib2].at[0]], rows[rb],
                                 gsem[rb])
        return carry

    lax.fori_loop(0, CPW // IBUF, step, 0)
    plsc.subcore_barrier()
    pltpu.sync_copy(h_shared.at[pl.ds(tile_base, ROWS_PER_TILE)],
                    out_hbm.at[c, pl.ds(tile_base, ROWS_PER_TILE)])


def _selfloop_body(x_ref, lw_ref, bias_ref, out_ref):
    out_ref[...] = lax.dot_general(
        x_ref[...], lw_ref[...], (((1,), (1,)), ((), ())),
        preferred_element_type=jnp.float32) + bias_ref[0]


def _combine_body(h0_ref, parts_ref, out_ref):
    out_ref[...] = parts_ref[0] + parts_ref[1] + h0_ref[...]


def kernel(x, edge_index, etypes, weight, w_comp, h_bias, loop_weight):
    src = edge_index[0]
    dst = edge_index[1]
    idx = etypes.astype(jnp.int32) * N + src.astype(jnp.int32)
    pad = E_PAD - E
    idx_p = jnp.concatenate([idx, jnp.zeros((pad,), jnp.int32)]).reshape(
        NW, CPW, 1, CHUNK)
    dst_p = jnp.concatenate([dst.astype(jnp.int32),
                             jnp.full((pad,), N, jnp.int32)]).reshape(
        NW, CPW, 1, CHUNK)
    idd_p = jnp.concatenate([idx_p, dst_p], axis=2)

    xw = pl.pallas_call(
        _xw_body,
        grid=(N // BN, NUM_RELS),
        in_specs=[
            pl.BlockSpec(memory_space=pltpu.SMEM),
            pl.BlockSpec((NUM_BASES, IN_FEAT, OUT_FEAT), lambda nb, r: (0, 0, 0)),
            pl.BlockSpec((BN, IN_FEAT), lambda nb, r: (nb, 0)),
        ],
        out_specs=pl.BlockSpec((1, BN, OUT_FEAT), lambda nb, r: (r, nb, 0)),
        out_shape=jax.ShapeDtypeStruct((NUM_RELS, N, OUT_FEAT), jnp.float32),
    )(w_comp, weight, x)
    xw_flat = xw.reshape(NUM_RELS * N, OUT_FEAT)

    zeros_rows = jnp.zeros((ROWS_PER_TILE, OUT_FEAT), jnp.float32)

    mesh = plsc.VectorSubcoreMesh(core_axis_name="c", subcore_axis_name="s",
                                  num_cores=NC, num_subcores=NS)
    parts = pl.kernel(
        _sc_body,
        out_type=jax.ShapeDtypeStruct((NC, NPAD, OUT_FEAT), jnp.float32),
        mesh=mesh,
        scratch_types=[
            pltpu.VMEM((2, CHUNK), jnp.int32),
            pltpu.VMEM((2, CHUNK), jnp.int32),
            pltpu.VMEM((2, CHUNK), jnp.int32),
            pltpu.VMEM((2, CHUNK), jnp.int32),
            pltpu.VMEM((CHUNK, OUT_FEAT), jnp.float32),
            pltpu.VMEM((CHUNK, OUT_FEAT), jnp.float32),
            pltpu.VMEM_SHARED((NPAD, OUT_FEAT), jnp.float32),
            pltpu.SemaphoreType.DMA,
            pltpu.SemaphoreType.DMA,
            pltpu.SemaphoreType.DMA,
            pltpu.SemaphoreType.DMA,
            pltpu.SemaphoreType.DMA,
            pltpu.SemaphoreType.DMA,
        ],
    )(xw_flat, idd_p, zeros_rows)

    h0 = pl.pallas_call(
        _selfloop_body,
        grid=(N // BN,),
        in_specs=[
            pl.BlockSpec((BN, IN_FEAT), lambda nb: (nb, 0)),
            pl.BlockSpec((OUT_FEAT, IN_FEAT), lambda nb: (0, 0)),
            pl.BlockSpec((1, OUT_FEAT), lambda nb: (0, 0)),
        ],
        out_specs=pl.BlockSpec((BN, OUT_FEAT), lambda nb: (nb, 0)),
        out_shape=jax.ShapeDtypeStruct((N, OUT_FEAT), jnp.float32),
    )(x, loop_weight, h_bias.reshape(1, OUT_FEAT))

    h = pl.pallas_call(
        _combine_body,
        grid=(N // BN,),
        in_specs=[
            pl.BlockSpec((BN, OUT_FEAT), lambda nb: (nb, 0)),
            pl.BlockSpec((NC, BN, OUT_FEAT), lambda nb: (0, nb, 0)),
        ],
        out_specs=pl.BlockSpec((BN, OUT_FEAT), lambda nb: (nb, 0)),
        out_shape=jax.ShapeDtypeStruct((N, OUT_FEAT), jnp.float32),
    )(h0, parts)
    return h


# speedup vs baseline: 2.0131x; 2.0131x over previous
import jax
import jax.numpy as jnp
from jax import lax
from jax.experimental import pallas as pl
from jax.experimental.pallas import tpu as pltpu
from jax.experimental.pallas import tpu_sc as plsc

N = 10000
E = 320000
IN_FEAT = 128
OUT_FEAT = 128
NUM_RELS = 32
NUM_BASES = 8

NC = 2
NS = 16
NW = NC * NS
CHUNK = 128
NBUF = 2
IBUF = 2 * NBUF
CPW = -(-E // (CHUNK * NW * IBUF)) * IBUF
NCHUNKS = CPW * NW
E_PAD = NCHUNKS * CHUNK
NPAD = 10240
ROWS_PER_TILE = NPAD // NS
BN = 1000


def _xw_body(w_comp_ref, weight_ref, x_ref, out_ref):
    r = pl.program_id(1)
    w = w_comp_ref[r, 0] * weight_ref[0]
    for b in range(1, NUM_BASES):
        w = w + w_comp_ref[r, b] * weight_ref[b]
    out_ref[0] = jnp.dot(x_ref[...], w, preferred_element_type=jnp.float32)


def _sc_body(xw_hbm, idd_hbm, zeros_hbm, out_hbm,
             idd0, idd1, idd2, idd3, rows0, rows1, h_shared,
             isem0, isem1, isem2, isem3, gsem0, gsem1):
    idd = (idd0, idd1, idd2, idd3)
    isem = (isem0, isem1, isem2, isem3)
    rows = (rows0, rows1)
    gsem = (gsem0, gsem1)
    c = lax.axis_index("c")
    s = lax.axis_index("s")
    wid = s * NC + c
    tile_base = s * ROWS_PER_TILE
    pltpu.sync_copy(zeros_hbm, h_shared.at[pl.ds(tile_base, ROWS_PER_TILE)])
    plsc.subcore_barrier()

    for j in range(IBUF):
        pltpu.async_copy(idd_hbm.at[wid, j], idd[j], isem[j])
    for j in range(NBUF):
        pltpu.make_async_copy(idd_hbm.at[wid, j], idd[j], isem[j]).wait()
        pltpu.async_copy(xw_hbm.at[idd[j].at[0]], rows[j], gsem[j])

    def step(j0, carry):
        for u in range(IBUF):
            j = j0 * IBUF + u
            ib, rb = u, u % NBUF
            pltpu.make_async_copy(xw_hbm.at[idd[ib].at[0]], rows[rb],
                                  gsem[rb]).wait()
            pltpu.sync_copy(rows[rb], h_shared.at[idd[ib].at[1]], add=True)

            @pl.when(j + IBUF < CPW)
            def _():
                pltpu.async_copy(idd_hbm.at[wid, j + IBUF], idd[ib], isem[ib])

            @pl.when(j + NBUF < CPW)
            def _():
                ib2 = (u + NBUF) % IBUF
                pltpu.make_async_copy(idd_hbm.at[wid, j + NBUF], idd[ib2],
                                      isem[ib2]).wait()
                pltpu.async_copy(xw_hbm.at[idd[ib2].at[0]], rows[rb],
                                 gsem[rb])
        return carry

    lax.fori_loop(0, CPW // IBUF, step, 0)
    plsc.subcore_barrier()
    pltpu.sync_copy(h_shared.at[pl.ds(tile_base, ROWS_PER_TILE)],
                    out_hbm.at[c, pl.ds(tile_base, ROWS_PER_TILE)])


def _selfloop_body(x_ref, lw_ref, bias_ref, out_ref):
    out_ref[...] = lax.dot_general(
        x_ref[...], lw_ref[...], (((1,), (1,)), ((), ())),
        preferred_element_type=jnp.float32) + bias_ref[0]


def _combine_body(h0_ref, parts_ref, out_ref):
    out_ref[...] = parts_ref[0] + parts_ref[1] + h0_ref[...]


def kernel(x, edge_index, etypes, weight, w_comp, h_bias, loop_weight):
    src = edge_index[0]
    dst = edge_index[1]
    idx = etypes.astype(jnp.int32) * N + src.astype(jnp.int32)
    pad = E_PAD - E
    pad_r = jnp.arange(pad, dtype=jnp.int32)
    idx_p = jnp.concatenate([idx, pad_r % 4096]).reshape(
        NW, CPW, 1, CHUNK)
    dst_p = jnp.concatenate([dst.astype(jnp.int32),
                             N + pad_r % (NPAD - N)]).reshape(
        NW, CPW, 1, CHUNK)
    idd_p = jnp.concatenate([idx_p, dst_p], axis=2)

    xw = pl.pallas_call(
        _xw_body,
        grid=(N // BN, NUM_RELS),
        in_specs=[
            pl.BlockSpec(memory_space=pltpu.SMEM),
            pl.BlockSpec((NUM_BASES, IN_FEAT, OUT_FEAT), lambda nb, r: (0, 0, 0)),
            pl.BlockSpec((BN, IN_FEAT), lambda nb, r: (nb, 0)),
        ],
        out_specs=pl.BlockSpec((1, BN, OUT_FEAT), lambda nb, r: (r, nb, 0)),
        out_shape=jax.ShapeDtypeStruct((NUM_RELS, N, OUT_FEAT), jnp.float32),
    )(w_comp, weight, x)
    xw_flat = xw.reshape(NUM_RELS * N, OUT_FEAT)

    zeros_rows = jnp.zeros((ROWS_PER_TILE, OUT_FEAT), jnp.float32)

    mesh = plsc.VectorSubcoreMesh(core_axis_name="c", subcore_axis_name="s",
                                  num_cores=NC, num_subcores=NS)
    parts = pl.kernel(
        _sc_body,
        out_type=jax.ShapeDtypeStruct((NC, NPAD, OUT_FEAT), jnp.float32),
        mesh=mesh,
        scratch_types=[
            pltpu.VMEM((2, CHUNK), jnp.int32),
            pltpu.VMEM((2, CHUNK), jnp.int32),
            pltpu.VMEM((2, CHUNK), jnp.int32),
            pltpu.VMEM((2, CHUNK), jnp.int32),
            pltpu.VMEM((CHUNK, OUT_FEAT), jnp.float32),
            pltpu.VMEM((CHUNK, OUT_FEAT), jnp.float32),
            pltpu.VMEM_SHARED((NPAD, OUT_FEAT), jnp.float32),
            pltpu.SemaphoreType.DMA,
            pltpu.SemaphoreType.DMA,
            pltpu.SemaphoreType.DMA,
            pltpu.SemaphoreType.DMA,
            pltpu.SemaphoreType.DMA,
            pltpu.SemaphoreType.DMA,
        ],
    )(xw_flat, idd_p, zeros_rows)

    h0 = pl.pallas_call(
        _selfloop_body,
        grid=(N // BN,),
        in_specs=[
            pl.BlockSpec((BN, IN_FEAT), lambda nb: (nb, 0)),
            pl.BlockSpec((OUT_FEAT, IN_FEAT), lambda nb: (0, 0)),
            pl.BlockSpec((1, OUT_FEAT), lambda nb: (0, 0)),
        ],
        out_specs=pl.BlockSpec((BN, OUT_FEAT), lambda nb: (nb, 0)),
        out_shape=jax.ShapeDtypeStruct((N, OUT_FEAT), jnp.float32),
    )(x, loop_weight, h_bias.reshape(1, OUT_FEAT))

    h = pl.pallas_call(
        _combine_body,
        grid=(N // BN,),
        in_specs=[
            pl.BlockSpec((BN, OUT_FEAT), lambda nb: (nb, 0)),
            pl.BlockSpec((NC, BN, OUT_FEAT), lambda nb: (0, nb, 0)),
        ],
        out_specs=pl.BlockSpec((BN, OUT_FEAT), lambda nb: (nb, 0)),
        out_shape=jax.ShapeDtypeStruct((N, OUT_FEAT), jnp.float32),
    )(h0, parts)
    return h


# speedup vs baseline: 3.2971x; 1.6378x over previous
import jax
import jax.numpy as jnp
from jax import lax
from jax.experimental import pallas as pl
from jax.experimental.pallas import tpu as pltpu
from jax.experimental.pallas import tpu_sc as plsc

N = 10000
E = 320000
IN_FEAT = 128
OUT_FEAT = 128
NUM_RELS = 32
NUM_BASES = 8

NC = 2
NS = 16
NW = NC * NS
CHUNK = 128
NBUF = 2
IBUF = 2 * NBUF
CPW = -(-E // (CHUNK * NW * IBUF)) * IBUF
NCHUNKS = CPW * NW
E_PAD = NCHUNKS * CHUNK
NPAD = 10240
ROWS_PER_TILE = NPAD // NS
BN = 1000


def _xw_body(w_comp_ref, weight_ref, x_ref, out_ref):
    r = pl.program_id(0)
    w = w_comp_ref[r, 0] * weight_ref[0]
    for b in range(1, NUM_BASES):
        w = w + w_comp_ref[r, b] * weight_ref[b]
    out_ref[0] = jnp.dot(x_ref[...], w, preferred_element_type=jnp.float32)


def _sc_body(xw_hbm, idd_hbm, zeros_hbm, out_hbm,
             idd0, idd1, idd2, idd3, rows0, rows1, h_shared,
             isem0, isem1, isem2, isem3, gsem0, gsem1):
    idd = (idd0, idd1, idd2, idd3)
    isem = (isem0, isem1, isem2, isem3)
    rows = (rows0, rows1)
    gsem = (gsem0, gsem1)
    c = lax.axis_index("c")
    s = lax.axis_index("s")
    wid = s * NC + c
    tile_base = s * ROWS_PER_TILE
    pltpu.sync_copy(zeros_hbm, h_shared.at[pl.ds(tile_base, ROWS_PER_TILE)])
    plsc.subcore_barrier()

    for j in range(IBUF):
        pltpu.async_copy(idd_hbm.at[wid, j], idd[j], isem[j])
    for j in range(NBUF):
        pltpu.make_async_copy(idd_hbm.at[wid, j], idd[j], isem[j]).wait()
        pltpu.async_copy(xw_hbm.at[idd[j].at[0]], rows[j], gsem[j])

    def step(j0, carry):
        for u in range(IBUF):
            j = j0 * IBUF + u
            ib, rb = u, u % NBUF
            pltpu.make_async_copy(xw_hbm.at[idd[ib].at[0]], rows[rb],
                                  gsem[rb]).wait()
            pltpu.sync_copy(rows[rb], h_shared.at[idd[ib].at[1]], add=True)

            @pl.when(j + IBUF < CPW)
            def _():
                pltpu.async_copy(idd_hbm.at[wid, j + IBUF], idd[ib], isem[ib])

            @pl.when(j + NBUF < CPW)
            def _():
                ib2 = (u + NBUF) % IBUF
                pltpu.make_async_copy(idd_hbm.at[wid, j + NBUF], idd[ib2],
                                      isem[ib2]).wait()
                pltpu.async_copy(xw_hbm.at[idd[ib2].at[0]], rows[rb],
                                 gsem[rb])
        return carry

    lax.fori_loop(0, CPW // IBUF, step, 0)
    plsc.subcore_barrier()
    pltpu.sync_copy(h_shared.at[pl.ds(tile_base, ROWS_PER_TILE)],
                    out_hbm.at[c, pl.ds(tile_base, ROWS_PER_TILE)])


def _selfloop_body(x_ref, lw_ref, bias_ref, out_ref):
    out_ref[...] = lax.dot_general(
        x_ref[...], lw_ref[...], (((1,), (1,)), ((), ())),
        preferred_element_type=jnp.float32) + bias_ref[0]


def _combine_body(h0_ref, parts_ref, out_ref):
    out_ref[...] = parts_ref[0] + parts_ref[1] + h0_ref[...]


def kernel(x, edge_index, etypes, weight, w_comp, h_bias, loop_weight):
    src = edge_index[0]
    dst = edge_index[1]
    idx = etypes.astype(jnp.int32) * N + src.astype(jnp.int32)
    pad = E_PAD - E
    pad_r = jnp.arange(pad, dtype=jnp.int32)
    idx_p = jnp.concatenate([idx, pad_r % 4096]).reshape(
        NW, CPW, 1, CHUNK)
    dst_p = jnp.concatenate([dst.astype(jnp.int32),
                             N + pad_r % (NPAD - N)]).reshape(
        NW, CPW, 1, CHUNK)
    idd_p = jnp.concatenate([idx_p, dst_p], axis=2)

    xw = pl.pallas_call(
        _xw_body,
        grid=(NUM_RELS,),
        in_specs=[
            pl.BlockSpec(memory_space=pltpu.SMEM),
            pl.BlockSpec((NUM_BASES, IN_FEAT, OUT_FEAT), lambda r: (0, 0, 0)),
            pl.BlockSpec((N, IN_FEAT), lambda r: (0, 0)),
        ],
        out_specs=pl.BlockSpec((1, N, OUT_FEAT), lambda r: (r, 0, 0)),
        out_shape=jax.ShapeDtypeStruct((NUM_RELS, N, OUT_FEAT), jnp.float32),
    )(w_comp, weight, x)
    xw_flat = xw.reshape(NUM_RELS * N, OUT_FEAT)

    zeros_rows = jnp.zeros((ROWS_PER_TILE, OUT_FEAT), jnp.float32)

    mesh = plsc.VectorSubcoreMesh(core_axis_name="c", subcore_axis_name="s",
                                  num_cores=NC, num_subcores=NS)
    parts = pl.kernel(
        _sc_body,
        out_type=jax.ShapeDtypeStruct((NC, NPAD, OUT_FEAT), jnp.float32),
        mesh=mesh,
        scratch_types=[
            pltpu.VMEM((2, CHUNK), jnp.int32),
            pltpu.VMEM((2, CHUNK), jnp.int32),
            pltpu.VMEM((2, CHUNK), jnp.int32),
            pltpu.VMEM((2, CHUNK), jnp.int32),
            pltpu.VMEM((CHUNK, OUT_FEAT), jnp.float32),
            pltpu.VMEM((CHUNK, OUT_FEAT), jnp.float32),
            pltpu.VMEM_SHARED((NPAD, OUT_FEAT), jnp.float32),
            pltpu.SemaphoreType.DMA,
            pltpu.SemaphoreType.DMA,
            pltpu.SemaphoreType.DMA,
            pltpu.SemaphoreType.DMA,
            pltpu.SemaphoreType.DMA,
            pltpu.SemaphoreType.DMA,
        ],
    )(xw_flat, idd_p, zeros_rows)

    h0 = pl.pallas_call(
        _selfloop_body,
        grid=(N // BN,),
        in_specs=[
            pl.BlockSpec((BN, IN_FEAT), lambda nb: (nb, 0)),
            pl.BlockSpec((OUT_FEAT, IN_FEAT), lambda nb: (0, 0)),
            pl.BlockSpec((1, OUT_FEAT), lambda nb: (0, 0)),
        ],
        out_specs=pl.BlockSpec((BN, OUT_FEAT), lambda nb: (nb, 0)),
        out_shape=jax.ShapeDtypeStruct((N, OUT_FEAT), jnp.float32),
    )(x, loop_weight, h_bias.reshape(1, OUT_FEAT))

    h = pl.pallas_call(
        _combine_body,
        grid=(N // BN,),
        in_specs=[
            pl.BlockSpec((BN, OUT_FEAT), lambda nb: (nb, 0)),
            pl.BlockSpec((NC, BN, OUT_FEAT), lambda nb: (0, nb, 0)),
        ],
        out_specs=pl.BlockSpec((BN, OUT_FEAT), lambda nb: (nb, 0)),
        out_shape=jax.ShapeDtypeStruct((N, OUT_FEAT), jnp.float32),
    )(h0, parts)
    return h


# speedup vs baseline: 3.3054x; 1.0025x over previous
import jax
import jax.numpy as jnp
from jax import lax
from jax.experimental import pallas as pl
from jax.experimental.pallas import tpu as pltpu
from jax.experimental.pallas import tpu_sc as plsc

N = 10000
E = 320000
IN_FEAT = 128
OUT_FEAT = 128
NUM_RELS = 32
NUM_BASES = 8

NC = 2
NS = 16
NW = NC * NS
CHUNK = 128
NBUF = 2
IBUF = 2 * NBUF
CPW = -(-E // (CHUNK * NW * IBUF)) * IBUF
NCHUNKS = CPW * NW
E_PAD = NCHUNKS * CHUNK
NPAD = 10240
ROWS_PER_TILE = NPAD // NS
BN = 1000


def _xw_body(w_comp_ref, weight_ref, x_ref, out_ref):
    r = pl.program_id(0)
    w = w_comp_ref[r, 0] * weight_ref[0]
    for b in range(1, NUM_BASES):
        w = w + w_comp_ref[r, b] * weight_ref[b]
    out_ref[0] = jnp.dot(x_ref[...], w, preferred_element_type=jnp.float32)


def _fetch_idd(idx_hbm, dst_hbm, wid, j, slot, sem):
    pltpu.async_copy(idx_hbm.at[wid, j], slot.at[0], sem)
    pltpu.async_copy(dst_hbm.at[wid, j], slot.at[1], sem)


def _wait_idd(idx_hbm, dst_hbm, wid, j, slot, sem):
    pltpu.make_async_copy(idx_hbm.at[wid, j], slot.at[0], sem).wait()
    pltpu.make_async_copy(dst_hbm.at[wid, j], slot.at[1], sem).wait()


def _sc_body(xw_hbm, idx_hbm, dst_hbm, zeros_hbm, out_hbm, *scratch):
    idd = scratch[:IBUF]
    rows = scratch[IBUF:IBUF + NBUF]
    h_shared = scratch[IBUF + NBUF]
    isem = scratch[IBUF + NBUF + 1:IBUF + NBUF + 1 + IBUF]
    gsem = scratch[IBUF + NBUF + 1 + IBUF:]
    c = lax.axis_index("c")
    s = lax.axis_index("s")
    wid = s * NC + c
    tile_base = s * ROWS_PER_TILE
    pltpu.sync_copy(zeros_hbm, h_shared.at[pl.ds(tile_base, ROWS_PER_TILE)])
    plsc.subcore_barrier()

    for j in range(IBUF):
        _fetch_idd(idx_hbm, dst_hbm, wid, j, idd[j], isem[j])
    for j in range(NBUF):
        _wait_idd(idx_hbm, dst_hbm, wid, j, idd[j], isem[j])
        pltpu.async_copy(xw_hbm.at[idd[j].at[0]], rows[j], gsem[j])

    def step(j0, carry):
        for u in range(IBUF):
            j = j0 * IBUF + u
            ib, rb = u, u % NBUF
            pltpu.make_async_copy(xw_hbm.at[idd[ib].at[0]], rows[rb],
                                  gsem[rb]).wait()
            pltpu.sync_copy(rows[rb], h_shared.at[idd[ib].at[1]], add=True)

            @pl.when(j + IBUF < CPW)
            def _():
                _fetch_idd(idx_hbm, dst_hbm, wid, j + IBUF, idd[ib], isem[ib])

            @pl.when(j + NBUF < CPW)
            def _():
                ib2 = (u + NBUF) % IBUF
                _wait_idd(idx_hbm, dst_hbm, wid, j + NBUF, idd[ib2], isem[ib2])
                pltpu.async_copy(xw_hbm.at[idd[ib2].at[0]], rows[rb],
                                 gsem[rb])
        return carry

    lax.fori_loop(0, CPW // IBUF, step, 0)
    plsc.subcore_barrier()
    pltpu.sync_copy(h_shared.at[pl.ds(tile_base, ROWS_PER_TILE)],
                    out_hbm.at[c, pl.ds(tile_base, ROWS_PER_TILE)])


def _selfloop_body(x_ref, lw_ref, bias_ref, out_ref):
    out_ref[...] = lax.dot_general(
        x_ref[...], lw_ref[...], (((1,), (1,)), ((), ())),
        preferred_element_type=jnp.float32) + bias_ref[0]


def _combine_body(h0_ref, parts_ref, out_ref):
    out_ref[...] = parts_ref[0] + parts_ref[1] + h0_ref[...]


def kernel(x, edge_index, etypes, weight, w_comp, h_bias, loop_weight):
    src = edge_index[0]
    dst = edge_index[1]
    idx = etypes.astype(jnp.int32) * N + src.astype(jnp.int32)
    pad = E_PAD - E
    pad_r = jnp.arange(pad, dtype=jnp.int32)
    idx_p = jnp.concatenate([idx, pad_r % 4096]).reshape(NW, CPW, CHUNK)
    dst_p = jnp.concatenate([dst.astype(jnp.int32),
                             N + pad_r % (NPAD - N)]).reshape(NW, CPW, CHUNK)

    xw = pl.pallas_call(
        _xw_body,
        grid=(NUM_RELS,),
        in_specs=[
            pl.BlockSpec(memory_space=pltpu.SMEM),
            pl.BlockSpec((NUM_BASES, IN_FEAT, OUT_FEAT), lambda r: (0, 0, 0)),
            pl.BlockSpec((N, IN_FEAT), lambda r: (0, 0)),
        ],
        out_specs=pl.BlockSpec((1, N, OUT_FEAT), lambda r: (r, 0, 0)),
        out_shape=jax.ShapeDtypeStruct((NUM_RELS, N, OUT_FEAT), jnp.float32),
    )(w_comp, weight, x)
    xw_flat = xw.reshape(NUM_RELS * N, OUT_FEAT)

    zeros_rows = jnp.zeros((ROWS_PER_TILE, OUT_FEAT), jnp.float32)

    mesh = plsc.VectorSubcoreMesh(core_axis_name="c", subcore_axis_name="s",
                                  num_cores=NC, num_subcores=NS)
    parts = pl.kernel(
        _sc_body,
        out_type=jax.ShapeDtypeStruct((NC, NPAD, OUT_FEAT), jnp.float32),
        mesh=mesh,
        scratch_types=(
            [pltpu.VMEM((2, CHUNK), jnp.int32)] * IBUF
            + [pltpu.VMEM((CHUNK, OUT_FEAT), jnp.float32)] * NBUF
            + [pltpu.VMEM_SHARED((NPAD, OUT_FEAT), jnp.float32)]
            + [pltpu.SemaphoreType.DMA] * (IBUF + NBUF)
        ),
    )(xw_flat, idx_p, dst_p, zeros_rows)

    h0 = pl.pallas_call(
        _selfloop_body,
        grid=(N // BN,),
        in_specs=[
            pl.BlockSpec((BN, IN_FEAT), lambda nb: (nb, 0)),
            pl.BlockSpec((OUT_FEAT, IN_FEAT), lambda nb: (0, 0)),
            pl.BlockSpec((1, OUT_FEAT), lambda nb: (0, 0)),
        ],
        out_specs=pl.BlockSpec((BN, OUT_FEAT), lambda nb: (nb, 0)),
        out_shape=jax.ShapeDtypeStruct((N, OUT_FEAT), jnp.float32),
    )(x, loop_weight, h_bias.reshape(1, OUT_FEAT))

    h = pl.pallas_call(
        _combine_body,
        grid=(N // BN,),
        in_specs=[
            pl.BlockSpec((BN, OUT_FEAT), lambda nb: (nb, 0)),
            pl.BlockSpec((NC, BN, OUT_FEAT), lambda nb: (0, nb, 0)),
        ],
        out_specs=pl.BlockSpec((BN, OUT_FEAT), lambda nb: (nb, 0)),
        out_shape=jax.ShapeDtypeStruct((N, OUT_FEAT), jnp.float32),
    )(h0, parts)
    return h
